# baseline (speedup 1.0000x reference)
import functools

import jax
import jax.numpy as jnp
from jax import lax
from jax.experimental import pallas as pl
from jax.experimental.pallas import tpu as pltpu
from jax.experimental.pallas import tpu_sc as plsc

_CHUNK = 32


def kernel(image_ids, embeddings_weight):
    (B,) = image_ids.shape
    V, D = embeddings_weight.shape
    info = plsc.get_sparse_core_info()
    NC, NS = info.num_cores, info.num_subcores
    NW = NC * NS
    assert B % (NW * _CHUNK) == 0
    b_per_w = B // NW
    n_chunks = b_per_w // _CHUNK

    mesh = plsc.VectorSubcoreMesh(core_axis_name="c", subcore_axis_name="s")

    @functools.partial(
        pl.kernel,
        mesh=mesh,
        out_type=jax.ShapeDtypeStruct((B, D), jnp.float32),
        scratch_types=[
            pltpu.VMEM((b_per_w,), jnp.int32),
            pltpu.VMEM((b_per_w, D), jnp.float32),
            pltpu.SemaphoreType.DMA,
        ],
    )
    def gather_kernel(idx_hbm, table_hbm, out_hbm, idx_v, rows_v, sem):
        wid = lax.axis_index("s") * NC + lax.axis_index("c")
        base = wid * b_per_w
        with jax.named_scope("stage_idx"):
            pltpu.sync_copy(idx_hbm.at[pl.ds(base, b_per_w)], idx_v)

        with jax.named_scope("fire_gathers"):

            @plsc.parallel_loop(0, n_chunks)
            def _fire(c):
                off = c * _CHUNK
                idx_vec = idx_v[pl.ds(off, _CHUNK)]
                for j in range(_CHUNK):
                    pltpu.async_copy(
                        table_hbm.at[pl.ds(idx_vec[j], 1)],
                        rows_v.at[pl.ds(off, _CHUNK)].at[pl.ds(j, 1)],
                        sem,
                    )

        with jax.named_scope("drain_gathers"):

            @pl.loop(0, n_chunks)
            def _drain(c):
                for j in range(_CHUNK):
                    pltpu.make_async_copy(
                        table_hbm.at[pl.ds(0, 1)],
                        rows_v.at[pl.ds(0, _CHUNK)].at[pl.ds(j, 1)],
                        sem,
                    ).wait()

        with jax.named_scope("writeback"):
            pltpu.sync_copy(rows_v, out_hbm.at[pl.ds(base, b_per_w)])

    return gather_kernel(image_ids.astype(jnp.int32), embeddings_weight)
